# baseline (speedup 1.0000x reference)
import dataclasses
import functools

import jax
import jax.numpy as jnp
from jax import lax
from jax.experimental import pallas as pl
from jax.experimental.pallas import tpu as pltpu
from jax.experimental.pallas import tpu_sc as plsc

F32 = jnp.float32
I32 = jnp.int32

_NC = 2
_NS = 16
_NW = _NC * _NS
_K = 128

_PREC = lax.Precision.HIGHEST


def _silu(t):
    return t * jax.nn.sigmoid(t)


def _full16(v):
    return jnp.full((16,), v, dtype=I32)


def _sc_params(tc_tiling=True):
    cp = pltpu.CompilerParams()
    fields = pltpu.CompilerParams.__dataclass_fields__
    if "needs_layout_passes" in fields:
        cp = dataclasses.replace(cp, needs_layout_passes=False)
    if not tc_tiling and "use_tc_tiling_on_sc" in fields:
        cp = dataclasses.replace(cp, use_tc_tiling_on_sc=False)
    return cp


def _make_gather(E, N):
    n_chunks = E // _K
    base_cnt = n_chunks // _NW
    n_extra = n_chunks - base_cnt * _NW
    mesh = plsc.VectorSubcoreMesh(core_axis_name="c", subcore_axis_name="s")

    @functools.partial(
        pl.kernel,
        mesh=mesh,
        out_type=jax.ShapeDtypeStruct((E, 128), F32),
        scratch_types=[
            pltpu.VMEM((N, 4), F32),
            pltpu.VMEM((_K,), I32),
            pltpu.VMEM((_K,), I32),
            pltpu.VMEM((_K, 128), F32),
            pltpu.VMEM((_K, 128), F32),
            pltpu.SemaphoreType.DMA,
            pltpu.SemaphoreType.DMA,
        ],
        compiler_params=_sc_params(tc_tiling=False),
    )
    def gather_k(pab_hbm, x4_hbm, row_hbm, col_hbm, gpre_hbm,
                 xtab, rowi, coli, bufr, bufc, sema, semb):
        cid = lax.axis_index("c")
        sid = lax.axis_index("s")
        wid = sid * _NC + cid
        cnt = base_cnt + jnp.where(wid < n_extra, 1, 0)
        pltpu.sync_copy(x4_hbm, xtab)

        @pl.loop(0, base_cnt + 1)
        def _(i):
            @pl.when(i < cnt)
            def _():
                base = (wid + _NW * i) * _K
                pltpu.sync_copy(row_hbm.at[pl.ds(base, _K)], rowi)
                pltpu.sync_copy(col_hbm.at[pl.ds(base, _K)], coli)
                cpa = pltpu.async_copy(pab_hbm.at[rowi], bufr, sema)
                cpb = pltpu.async_copy(pab_hbm.at[coli], bufc, semb)
                cpa.wait()
                cpb.wait()

                @pl.loop(0, _K, step=16)
                def _(g):
                    rv = rowi[pl.ds(g, 16)]
                    cv = coli[pl.ds(g, 16)]
                    d0 = (plsc.load_gather(xtab, [rv, _full16(0)])
                          - plsc.load_gather(xtab, [cv, _full16(0)]))
                    d1 = (plsc.load_gather(xtab, [rv, _full16(1)])
                          - plsc.load_gather(xtab, [cv, _full16(1)]))
                    d2 = (plsc.load_gather(xtab, [rv, _full16(2)])
                          - plsc.load_gather(xtab, [cv, _full16(2)]))
                    rad = d0 * d0 + d1 * d1 + d2 * d2
                    rows = lax.iota(I32, 16) + g
                    plsc.store_scatter(bufr, [rows, _full16(64)], rad)
                    plsc.store_scatter(bufr, [rows, _full16(65)], d0)
                    plsc.store_scatter(bufr, [rows, _full16(66)], d1)
                    plsc.store_scatter(bufr, [rows, _full16(67)], d2)

                @pl.loop(0, _K)
                def _(e):
                    for c in (0, 16, 32, 48):
                        bufr[e, pl.ds(c, 16)] = (
                            bufr[e, pl.ds(c, 16)]
                            + bufc[e, pl.ds(c + 64, 16)])

                pltpu.sync_copy(bufr, gpre_hbm.at[pl.ds(base, _K)])

    return gather_k


def _make_scatter(E, N):
    n_chunks = E // _K
    base_cnt = n_chunks // _NW
    n_extra = n_chunks - base_cnt * _NW
    rows_per_tile = N // _NS
    zrows = rows_per_tile // 5
    Eq = E // 4
    mesh = plsc.VectorSubcoreMesh(core_axis_name="c", subcore_axis_name="s")

    @functools.partial(
        pl.kernel,
        mesh=mesh,
        out_type=jax.ShapeDtypeStruct((_NC, N, 128), F32),
        scratch_types=[
            pltpu.VMEM_SHARED((N, 128), F32),
            pltpu.VMEM((1, _K), I32),
            pltpu.VMEM((_K, 128), F32),
            pltpu.VMEM((zrows, 128), F32),
        ],
        compiler_params=_sc_params(tc_tiling=False),
    )
    def scatter_k(row_hbm, oe_hbm, part_hbm, acc, rowi, ebuf, zbuf):
        cid = lax.axis_index("c")
        sid = lax.axis_index("s")
        wid = sid * _NC + cid
        cnt = base_cnt + jnp.where(wid < n_extra, 1, 0)

        z16 = jnp.zeros((16,), F32)

        @pl.loop(0, zrows)
        def _(r):
            @pl.loop(0, 128, step=16)
            def _(c):
                zbuf[r, pl.ds(c, 16)] = z16

        @pl.loop(0, 5)
        def _(j):
            off = sid * rows_per_tile + j * zrows
            pltpu.sync_copy(zbuf, acc.at[pl.ds(off, zrows)])

        plsc.subcore_barrier()

        @pl.loop(0, base_cnt + 1)
        def _(i):
            @pl.when(i < cnt)
            def _():
                base = (wid + _NW * i) * _K
                q = base // Eq
                qoff = base - q * Eq
                pltpu.sync_copy(row_hbm.at[pl.ds(base, _K)], rowi.at[0])
                pltpu.sync_copy(oe_hbm.at[q, pl.ds(qoff, _K)], ebuf)
                pltpu.sync_copy(ebuf, acc.at[rowi.at[0]], add=True)

        plsc.subcore_barrier()

        off = sid * rows_per_tile
        pltpu.sync_copy(acc.at[pl.ds(off, rows_per_tile)],
                        part_hbm.at[cid, pl.ds(off, rows_per_tile)])

    return scatter_k


def _init_tc(his, emb_W, emb_b, W1a, W1b):
    N, D = his.shape
    BN = 1000

    def body(his_r, ew_r, eb_r, wa_r, wb_r, h_r, pab_r):
        h = jnp.dot(his_r[...], ew_r[...], precision=_PREC) + eb_r[...]
        h_r[...] = h
        pab_r[...] = jnp.concatenate(
            [jnp.dot(h, wa_r[...], precision=_PREC),
             jnp.dot(h, wb_r[...], precision=_PREC)], axis=1)

    return pl.pallas_call(
        body,
        grid=(N // BN,),
        in_specs=[
            pl.BlockSpec((BN, D), lambda i: (i, 0)),
            pl.BlockSpec((D, 64), lambda i: (0, 0)),
            pl.BlockSpec((1, 64), lambda i: (0, 0)),
            pl.BlockSpec((64, 64), lambda i: (0, 0)),
            pl.BlockSpec((64, 64), lambda i: (0, 0)),
        ],
        out_specs=[
            pl.BlockSpec((BN, 64), lambda i: (i, 0)),
            pl.BlockSpec((BN, 128), lambda i: (i, 0)),
        ],
        out_shape=[
            jax.ShapeDtypeStruct((N, 64), F32),
            jax.ShapeDtypeStruct((N, 128), F32),
        ],
    )(his, emb_W, emb_b, W1a, W1b)


def _eaproj_tc(edge_attr, W1d4, b1_4):
    E = edge_attr.shape[0]
    Eq = E // 4
    BE = 640
    nb = Eq // BE

    def body(e0_r, e1_r, e2_r, e3_r, w_r, b_r, o_r):
        ea_all = jnp.concatenate(
            [e0_r[...], e1_r[...], e2_r[...], e3_r[...]], axis=1)
        o_r[...] = jnp.dot(ea_all, w_r[...], precision=_PREC) + b_r[...]

    def mk(k):
        return pl.BlockSpec((BE, 16), lambda i, k=k: (i + k * nb, 0))

    return pl.pallas_call(
        body,
        grid=(nb,),
        in_specs=[
            mk(0), mk(1), mk(2), mk(3),
            pl.BlockSpec((64, 256), lambda i: (0, 0)),
            pl.BlockSpec((1, 256), lambda i: (0, 0)),
        ],
        out_specs=pl.BlockSpec((BE, 256), lambda i: (i, 0)),
        out_shape=jax.ShapeDtypeStruct((Eq, 256), F32),
    )(edge_attr, edge_attr, edge_attr, edge_attr, W1d4, b1_4)


def _edge_tc(gpre, eap, w1c, W2d4, b2_4, cW1d4, cb1_4, cW2_4):
    E = gpre.shape[0]
    Eq = E // 4
    BE = 640
    nb = Eq // BE

    def body(g0_r, g1_r, g2_r, g3_r, eap_r, w1c_r, w2_r, b2_r,
             cw1_r, cb1_r, cw2_r, oe_r):
        gs = [g0_r[...], g1_r[...], g2_r[...], g3_r[...]]
        eapv = eap_r[...]
        w1cv = w1c_r[...]
        pre_all = jnp.concatenate(
            [gs[k][:, 0:64] + gs[k][:, 64:65] * w1cv for k in range(4)],
            axis=1) + eapv
        m_all = _silu(jnp.dot(_silu(pre_all), w2_r[...], precision=_PREC)
                      + b2_r[...])
        t_all = _silu(jnp.dot(m_all, cw1_r[...], precision=_PREC)
                      + cb1_r[...])
        s = t_all * cw2_r[...]
        ones = jnp.ones((BE, 1), F32)
        zer = jnp.zeros((BE, 60), F32)
        for k in range(4):
            phi = jnp.sum(s[:, 64 * k:64 * k + 64], axis=1, keepdims=True)
            trans = gs[k][:, 65:68] * phi
            oe_r[k] = jnp.concatenate(
                [m_all[:, 64 * k:64 * k + 64], trans, ones, zer], axis=1)

    def mk(k):
        return pl.BlockSpec((BE, 128), lambda i, k=k: (i + k * nb, 0))

    return pl.pallas_call(
        body,
        grid=(nb,),
        in_specs=[
            mk(0), mk(1), mk(2), mk(3),
            pl.BlockSpec((BE, 256), lambda i: (i, 0)),
            pl.BlockSpec((1, 64), lambda i: (0, 0)),
            pl.BlockSpec((256, 256), lambda i: (0, 0)),
            pl.BlockSpec((1, 256), lambda i: (0, 0)),
            pl.BlockSpec((256, 256), lambda i: (0, 0)),
            pl.BlockSpec((1, 256), lambda i: (0, 0)),
            pl.BlockSpec((1, 256), lambda i: (0, 0)),
        ],
        out_specs=pl.BlockSpec((4, BE, 128), lambda i: (0, i, 0)),
        out_shape=jax.ShapeDtypeStruct((4, Eq, 128), F32),
    )(gpre, gpre, gpre, gpre, eap, w1c, W2d4, b2_4, cW1d4, cb1_4, cW2_4)


def _node_tc(part, h, x4, v4, vW1, vb1, vW2r, vb2, nW1a, nW1b, nb1,
             nW2, nb2, W1a, W1b):
    N = h.shape[0]
    BN = 1000

    def body(p_r, h_r, x4_r, v4_r, vw1_r, vb1_r, vw2_r, vb2_r,
             nwa_r, nwb_r, nb1_r, nw2_r, nb2_r, wa_r, wb_r,
             hn_r, xn_r, vn_r, pab_r):
        h = h_r[...]
        pv = p_r[...]
        ptot = pv[0] + pv[1]
        aggm = ptot[:, 0:64]
        agg4 = ptot[:, 64:68] / jnp.maximum(ptot[:, 67:68], 1.0)
        vs1 = _silu(jnp.dot(h, vw1_r[...], precision=_PREC) + vb1_r[...])
        vscale = jnp.sum(vs1 * vw2_r[...], axis=1, keepdims=True) + vb2_r[...]
        v4n = vscale * v4_r[...] + agg4
        x4n = x4_r[...] + v4n
        hn1 = _silu(jnp.dot(h, nwa_r[...], precision=_PREC)
                    + jnp.dot(aggm, nwb_r[...], precision=_PREC) + nb1_r[...])
        hnn = jnp.dot(hn1, nw2_r[...], precision=_PREC) + nb2_r[...]
        h2 = 2.0 * h + hnn
        hn_r[...] = h2
        xn_r[...] = x4n
        vn_r[...] = v4n
        pab_r[...] = jnp.concatenate(
            [jnp.dot(h2, wa_r[...], precision=_PREC),
             jnp.dot(h2, wb_r[...], precision=_PREC)], axis=1)

    w64 = pl.BlockSpec((64, 64), lambda i: (0, 0))
    b64 = pl.BlockSpec((1, 64), lambda i: (0, 0))
    return pl.pallas_call(
        body,
        grid=(N // BN,),
        in_specs=[
            pl.BlockSpec((2, BN, 128), lambda i: (0, i, 0)),
            pl.BlockSpec((BN, 64), lambda i: (i, 0)),
            pl.BlockSpec((BN, 4), lambda i: (i, 0)),
            pl.BlockSpec((BN, 4), lambda i: (i, 0)),
            w64, b64, b64,
            pl.BlockSpec((1, 1), lambda i: (0, 0)),
            w64, w64, b64, w64, b64, w64, w64,
        ],
        out_specs=[
            pl.BlockSpec((BN, 64), lambda i: (i, 0)),
            pl.BlockSpec((BN, 4), lambda i: (i, 0)),
            pl.BlockSpec((BN, 4), lambda i: (i, 0)),
            pl.BlockSpec((BN, 128), lambda i: (i, 0)),
        ],
        out_shape=[
            jax.ShapeDtypeStruct((N, 64), F32),
            jax.ShapeDtypeStruct((N, 4), F32),
            jax.ShapeDtypeStruct((N, 4), F32),
            jax.ShapeDtypeStruct((N, 128), F32),
        ],
    )(part, h, x4, v4, vW1, vb1, vW2r, vb2, nW1a, nW1b, nb1, nW2, nb2,
      W1a, W1b)


def kernel(his, x, v, edges, edge_attr, params):
    p = params
    N = his.shape[0]
    E = edge_attr.shape[0]
    row = edges[0]
    col = edges[1]

    W1 = p['e_W1']
    W1a, W1b = W1[0:64], W1[64:128]
    w1c = W1[128:129]
    W1d = W1[129:145]
    nW1 = p['n_W1']
    nW1a, nW1b = nW1[0:64], nW1[64:128]
    emb_b = p['emb_b'].reshape(1, 64)
    e_b1 = p['e_b1'].reshape(1, 64)
    e_b2 = p['e_b2'].reshape(1, 64)
    c_b1 = p['c_b1'].reshape(1, 64)
    cW2r = p['c_W2'].reshape(1, 64)
    v_b1 = p['v_b1'].reshape(1, 64)
    vW2r = p['v_W2'].reshape(1, 64)
    v_b2 = p['v_b2'].reshape(1, 1)
    n_b1 = p['n_b1'].reshape(1, 64)
    n_b2 = p['n_b2'].reshape(1, 64)

    x4 = jnp.pad(x, ((0, 0), (0, 1)))
    v4 = jnp.pad(v, ((0, 0), (0, 1)))

    from jax.scipy.linalg import block_diag
    W1d4 = block_diag(W1d, W1d, W1d, W1d)
    W2d4 = block_diag(*([p['e_W2']] * 4))
    cW1d4 = block_diag(*([p['c_W1']] * 4))
    b1_4 = jnp.tile(e_b1, (1, 4))
    b2_4 = jnp.tile(e_b2, (1, 4))
    cb1_4 = jnp.tile(c_b1, (1, 4))
    cW2_4 = jnp.tile(cW2r, (1, 4))

    gather_k = _make_gather(E, N)
    scatter_k = _make_scatter(E, N)

    h, pab = _init_tc(his, p['emb_W'], emb_b, W1a, W1b)
    eap = _eaproj_tc(edge_attr, W1d4, b1_4)

    for _ in range(3):
        gpre = gather_k(pab, x4, row, col)
        oe = _edge_tc(gpre, eap, w1c, W2d4, b2_4, cW1d4, cb1_4, cW2_4)
        part = scatter_k(row, oe)
        h, x4, v4, pab = _node_tc(part, h, x4, v4, p['v_W1'], v_b1,
                                  vW2r, v_b2, nW1a, nW1b, n_b1,
                                  p['n_W2'], n_b2, W1a, W1b)

    return (x4[:, :3], h, v4[:, :3])


# speedup vs baseline: 1.2486x; 1.2486x over previous
import dataclasses
import functools

import jax
import jax.numpy as jnp
from jax import lax
from jax.experimental import pallas as pl
from jax.experimental.pallas import tpu as pltpu
from jax.experimental.pallas import tpu_sc as plsc

F32 = jnp.float32
I32 = jnp.int32

_NC = 2
_NS = 16
_NW = _NC * _NS
_K = 128

_PREC = lax.Precision.HIGHEST


def _silu(t):
    return t * jax.nn.sigmoid(t)


def _split_bf16(w):
    hi = w.astype(jnp.bfloat16)
    lo = (w - hi.astype(F32)).astype(jnp.bfloat16)
    return hi, lo


def _dot3(a, b_hi, b_lo):
    a_hi = a.astype(jnp.bfloat16)
    a_lo = (a - a_hi.astype(F32)).astype(jnp.bfloat16)
    return (jnp.dot(a_hi, b_hi, preferred_element_type=F32)
            + (jnp.dot(a_lo, b_hi, preferred_element_type=F32)
               + jnp.dot(a_hi, b_lo, preferred_element_type=F32)))


def _full16(v):
    return jnp.full((16,), v, dtype=I32)


def _sc_params(tc_tiling=True):
    cp = pltpu.CompilerParams()
    fields = pltpu.CompilerParams.__dataclass_fields__
    if "needs_layout_passes" in fields:
        cp = dataclasses.replace(cp, needs_layout_passes=False)
    if not tc_tiling and "use_tc_tiling_on_sc" in fields:
        cp = dataclasses.replace(cp, use_tc_tiling_on_sc=False)
    return cp


def _make_gather(E, N):
    n_chunks = E // _K
    base_cnt = n_chunks // _NW
    n_extra = n_chunks - base_cnt * _NW
    mesh = plsc.VectorSubcoreMesh(core_axis_name="c", subcore_axis_name="s")

    @functools.partial(
        pl.kernel,
        mesh=mesh,
        out_type=jax.ShapeDtypeStruct((E, 128), F32),
        scratch_types=[
            pltpu.VMEM((N, 4), F32),
            pltpu.VMEM((_K,), I32),
            pltpu.VMEM((_K,), I32),
            pltpu.VMEM((_K, 128), F32),
            pltpu.VMEM((_K, 128), F32),
            pltpu.SemaphoreType.DMA,
            pltpu.SemaphoreType.DMA,
        ],
        compiler_params=_sc_params(tc_tiling=False),
    )
    def gather_k(pab_hbm, x4_hbm, row_hbm, col_hbm, gpre_hbm,
                 xtab, rowi, coli, bufr, bufc, sema, semb):
        cid = lax.axis_index("c")
        sid = lax.axis_index("s")
        wid = sid * _NC + cid
        cnt = base_cnt + jnp.where(wid < n_extra, 1, 0)
        pltpu.sync_copy(x4_hbm, xtab)

        @pl.loop(0, base_cnt + 1)
        def _(i):
            @pl.when(i < cnt)
            def _():
                base = (wid + _NW * i) * _K
                pltpu.sync_copy(row_hbm.at[pl.ds(base, _K)], rowi)
                pltpu.sync_copy(col_hbm.at[pl.ds(base, _K)], coli)
                cpa = pltpu.async_copy(pab_hbm.at[rowi], bufr, sema)
                cpb = pltpu.async_copy(pab_hbm.at[coli], bufc, semb)
                cpa.wait()
                cpb.wait()

                @pl.loop(0, _K, step=16)
                def _(g):
                    rv = rowi[pl.ds(g, 16)]
                    cv = coli[pl.ds(g, 16)]
                    d0 = (plsc.load_gather(xtab, [rv, _full16(0)])
                          - plsc.load_gather(xtab, [cv, _full16(0)]))
                    d1 = (plsc.load_gather(xtab, [rv, _full16(1)])
                          - plsc.load_gather(xtab, [cv, _full16(1)]))
                    d2 = (plsc.load_gather(xtab, [rv, _full16(2)])
                          - plsc.load_gather(xtab, [cv, _full16(2)]))
                    rad = d0 * d0 + d1 * d1 + d2 * d2
                    rows = lax.iota(I32, 16) + g
                    plsc.store_scatter(bufr, [rows, _full16(64)], rad)
                    plsc.store_scatter(bufr, [rows, _full16(65)], d0)
                    plsc.store_scatter(bufr, [rows, _full16(66)], d1)
                    plsc.store_scatter(bufr, [rows, _full16(67)], d2)

                @pl.loop(0, _K)
                def _(e):
                    for c in (0, 16, 32, 48):
                        bufr[e, pl.ds(c, 16)] = (
                            bufr[e, pl.ds(c, 16)]
                            + bufc[e, pl.ds(c + 64, 16)])

                pltpu.sync_copy(bufr, gpre_hbm.at[pl.ds(base, _K)])

    return gather_k


def _make_scatter(E, N):
    n_chunks = E // _K
    base_cnt = n_chunks // _NW
    n_extra = n_chunks - base_cnt * _NW
    rows_per_tile = N // _NS
    zrows = rows_per_tile // 5
    Eh = E // 2
    mesh = plsc.VectorSubcoreMesh(core_axis_name="c", subcore_axis_name="s")

    @functools.partial(
        pl.kernel,
        mesh=mesh,
        out_type=jax.ShapeDtypeStruct((_NC, N, 128), F32),
        scratch_types=[
            pltpu.VMEM_SHARED((N, 128), F32),
            pltpu.VMEM((1, _K), I32),
            pltpu.VMEM((_K, 128), F32),
            pltpu.VMEM((zrows, 128), F32),
        ],
        compiler_params=_sc_params(tc_tiling=False),
    )
    def scatter_k(row_hbm, oelo_hbm, oehi_hbm, part_hbm, acc, rowi, ebuf,
                  zbuf):
        cid = lax.axis_index("c")
        sid = lax.axis_index("s")
        wid = sid * _NC + cid
        cnt = base_cnt + jnp.where(wid < n_extra, 1, 0)

        z16 = jnp.zeros((16,), F32)

        @pl.loop(0, zrows)
        def _(r):
            @pl.loop(0, 128, step=16)
            def _(c):
                zbuf[r, pl.ds(c, 16)] = z16

        @pl.loop(0, 5)
        def _(j):
            off = sid * rows_per_tile + j * zrows
            pltpu.sync_copy(zbuf, acc.at[pl.ds(off, zrows)])

        plsc.subcore_barrier()

        @pl.loop(0, base_cnt + 1)
        def _(i):
            @pl.when(i < cnt)
            def _():
                base = (wid + _NW * i) * _K
                pltpu.sync_copy(row_hbm.at[pl.ds(base, _K)], rowi.at[0])

                @pl.when(base < Eh)
                def _():
                    pltpu.sync_copy(oelo_hbm.at[pl.ds(base, _K)], ebuf)

                @pl.when(base >= Eh)
                def _():
                    pltpu.sync_copy(oehi_hbm.at[pl.ds(base - Eh, _K)], ebuf)

                pltpu.sync_copy(ebuf, acc.at[rowi.at[0]], add=True)

        plsc.subcore_barrier()

        off = sid * rows_per_tile
        pltpu.sync_copy(acc.at[pl.ds(off, rows_per_tile)],
                        part_hbm.at[cid, pl.ds(off, rows_per_tile)])

    return scatter_k


def _init_tc(his, emb_W, emb_b, W1a, W1b):
    N, D = his.shape
    BN = 1000

    def body(his_r, ew_r, eb_r, wa_r, wb_r, h_r, pab_r):
        h = jnp.dot(his_r[...], ew_r[...], precision=_PREC) + eb_r[...]
        h_r[...] = h
        pab_r[...] = jnp.concatenate(
            [jnp.dot(h, wa_r[...], precision=_PREC),
             jnp.dot(h, wb_r[...], precision=_PREC)], axis=1)

    return pl.pallas_call(
        body,
        grid=(N // BN,),
        in_specs=[
            pl.BlockSpec((BN, D), lambda i: (i, 0)),
            pl.BlockSpec((D, 64), lambda i: (0, 0)),
            pl.BlockSpec((1, 64), lambda i: (0, 0)),
            pl.BlockSpec((64, 64), lambda i: (0, 0)),
            pl.BlockSpec((64, 64), lambda i: (0, 0)),
        ],
        out_specs=[
            pl.BlockSpec((BN, 64), lambda i: (i, 0)),
            pl.BlockSpec((BN, 128), lambda i: (i, 0)),
        ],
        out_shape=[
            jax.ShapeDtypeStruct((N, 64), F32),
            jax.ShapeDtypeStruct((N, 128), F32),
        ],
    )(his, emb_W, emb_b, W1a, W1b)


def _eaproj_tc(edge_attr, W1d, b1):
    E = edge_attr.shape[0]
    Eh = E // 2
    BE = 640
    nb = Eh // BE
    w_hi, w_lo = _split_bf16(W1d)

    def body(lo_r, hi_r, wh_r, wl_r, b_r, o_r):
        o_r[...] = jnp.concatenate(
            [_dot3(lo_r[...], wh_r[...], wl_r[...]) + b_r[...],
             _dot3(hi_r[...], wh_r[...], wl_r[...]) + b_r[...]],
            axis=1)

    return pl.pallas_call(
        body,
        grid=(nb,),
        in_specs=[
            pl.BlockSpec((BE, 16), lambda i: (i, 0)),
            pl.BlockSpec((BE, 16), lambda i: (i + nb, 0)),
            pl.BlockSpec((16, 64), lambda i: (0, 0)),
            pl.BlockSpec((16, 64), lambda i: (0, 0)),
            pl.BlockSpec((1, 64), lambda i: (0, 0)),
        ],
        out_specs=pl.BlockSpec((BE, 128), lambda i: (i, 0)),
        out_shape=jax.ShapeDtypeStruct((Eh, 128), F32),
    )(edge_attr, edge_attr, w_hi, w_lo, b1)


def _edge_tc(gpre, eap, w1c, W2, b2, cW1, cb1, cW2r):
    E = gpre.shape[0]
    Eh = E // 2
    BE = 640
    nb = Eh // BE
    w2_hi, w2_lo = _split_bf16(W2)
    cw1_hi, cw1_lo = _split_bf16(cW1)

    def half(gp, ea, w1c_v, w2h, w2l, b2_v, cw1h, cw1l, cb1_v, cw2_v):
        pre = gp[:, 0:64] + ea + gp[:, 64:65] * w1c_v
        m = _silu(_dot3(_silu(pre), w2h, w2l) + b2_v)
        t = _silu(_dot3(m, cw1h, cw1l) + cb1_v)
        phi = jnp.sum(t * cw2_v, axis=1, keepdims=True)
        trans = gp[:, 65:68] * phi
        return jnp.concatenate(
            [m, trans, jnp.ones((BE, 1), F32), jnp.zeros((BE, 60), F32)],
            axis=1)

    def body(glo_r, ghi_r, eap_r, w1c_r, w2h_r, w2l_r, b2_r,
             cw1h_r, cw1l_r, cb1_r, cw2_r, olo_r, ohi_r):
        eapv = eap_r[...]
        olo_r[...] = half(glo_r[...], eapv[:, 0:64], w1c_r[...],
                          w2h_r[...], w2l_r[...], b2_r[...],
                          cw1h_r[...], cw1l_r[...], cb1_r[...], cw2_r[...])
        ohi_r[...] = half(ghi_r[...], eapv[:, 64:128], w1c_r[...],
                          w2h_r[...], w2l_r[...], b2_r[...],
                          cw1h_r[...], cw1l_r[...], cb1_r[...], cw2_r[...])

    b64 = pl.BlockSpec((1, 64), lambda i: (0, 0))
    w64 = pl.BlockSpec((64, 64), lambda i: (0, 0))
    return pl.pallas_call(
        body,
        grid=(nb,),
        in_specs=[
            pl.BlockSpec((BE, 128), lambda i: (i, 0)),
            pl.BlockSpec((BE, 128), lambda i: (i + nb, 0)),
            pl.BlockSpec((BE, 128), lambda i: (i, 0)),
            b64, w64, w64, b64, w64, w64, b64, b64,
        ],
        out_specs=[
            pl.BlockSpec((BE, 128), lambda i: (i, 0)),
            pl.BlockSpec((BE, 128), lambda i: (i, 0)),
        ],
        out_shape=[
            jax.ShapeDtypeStruct((Eh, 128), F32),
            jax.ShapeDtypeStruct((Eh, 128), F32),
        ],
    )(gpre, gpre, eap, w1c, w2_hi, w2_lo, b2, cw1_hi, cw1_lo, cb1, cW2r)


def _node_tc(part, h, x4, v4, vW1, vb1, vW2r, vb2, nW1a, nW1b, nb1,
             nW2, nb2, W1a, W1b):
    N = h.shape[0]
    BN = 1000

    def body(p_r, h_r, x4_r, v4_r, vw1_r, vb1_r, vw2_r, vb2_r,
             nwa_r, nwb_r, nb1_r, nw2_r, nb2_r, wa_r, wb_r,
             hn_r, xn_r, vn_r, pab_r):
        h = h_r[...]
        pv = p_r[...]
        ptot = pv[0] + pv[1]
        aggm = ptot[:, 0:64]
        agg4 = ptot[:, 64:68] / jnp.maximum(ptot[:, 67:68], 1.0)
        vs1 = _silu(jnp.dot(h, vw1_r[...], precision=_PREC) + vb1_r[...])
        vscale = jnp.sum(vs1 * vw2_r[...], axis=1, keepdims=True) + vb2_r[...]
        v4n = vscale * v4_r[...] + agg4
        x4n = x4_r[...] + v4n
        hn1 = _silu(jnp.dot(h, nwa_r[...], precision=_PREC)
                    + jnp.dot(aggm, nwb_r[...], precision=_PREC) + nb1_r[...])
        hnn = jnp.dot(hn1, nw2_r[...], precision=_PREC) + nb2_r[...]
        h2 = 2.0 * h + hnn
        hn_r[...] = h2
        xn_r[...] = x4n
        vn_r[...] = v4n
        pab_r[...] = jnp.concatenate(
            [jnp.dot(h2, wa_r[...], precision=_PREC),
             jnp.dot(h2, wb_r[...], precision=_PREC)], axis=1)

    w64 = pl.BlockSpec((64, 64), lambda i: (0, 0))
    b64 = pl.BlockSpec((1, 64), lambda i: (0, 0))
    return pl.pallas_call(
        body,
        grid=(N // BN,),
        in_specs=[
            pl.BlockSpec((2, BN, 128), lambda i: (0, i, 0)),
            pl.BlockSpec((BN, 64), lambda i: (i, 0)),
            pl.BlockSpec((BN, 4), lambda i: (i, 0)),
            pl.BlockSpec((BN, 4), lambda i: (i, 0)),
            w64, b64, b64,
            pl.BlockSpec((1, 1), lambda i: (0, 0)),
            w64, w64, b64, w64, b64, w64, w64,
        ],
        out_specs=[
            pl.BlockSpec((BN, 64), lambda i: (i, 0)),
            pl.BlockSpec((BN, 4), lambda i: (i, 0)),
            pl.BlockSpec((BN, 4), lambda i: (i, 0)),
            pl.BlockSpec((BN, 128), lambda i: (i, 0)),
        ],
        out_shape=[
            jax.ShapeDtypeStruct((N, 64), F32),
            jax.ShapeDtypeStruct((N, 4), F32),
            jax.ShapeDtypeStruct((N, 4), F32),
            jax.ShapeDtypeStruct((N, 128), F32),
        ],
    )(part, h, x4, v4, vW1, vb1, vW2r, vb2, nW1a, nW1b, nb1, nW2, nb2,
      W1a, W1b)


def kernel(his, x, v, edges, edge_attr, params):
    p = params
    N = his.shape[0]
    E = edge_attr.shape[0]
    row = edges[0]
    col = edges[1]

    W1 = p['e_W1']
    W1a, W1b = W1[0:64], W1[64:128]
    w1c = W1[128:129]
    W1d = W1[129:145]
    nW1 = p['n_W1']
    nW1a, nW1b = nW1[0:64], nW1[64:128]
    emb_b = p['emb_b'].reshape(1, 64)
    e_b1 = p['e_b1'].reshape(1, 64)
    e_b2 = p['e_b2'].reshape(1, 64)
    c_b1 = p['c_b1'].reshape(1, 64)
    cW2r = p['c_W2'].reshape(1, 64)
    v_b1 = p['v_b1'].reshape(1, 64)
    vW2r = p['v_W2'].reshape(1, 64)
    v_b2 = p['v_b2'].reshape(1, 1)
    n_b1 = p['n_b1'].reshape(1, 64)
    n_b2 = p['n_b2'].reshape(1, 64)

    x4 = jnp.pad(x, ((0, 0), (0, 1)))
    v4 = jnp.pad(v, ((0, 0), (0, 1)))

    gather_k = _make_gather(E, N)
    scatter_k = _make_scatter(E, N)

    h, pab = _init_tc(his, p['emb_W'], emb_b, W1a, W1b)
    eap = _eaproj_tc(edge_attr, W1d, e_b1)

    for _ in range(3):
        gpre = gather_k(pab, x4, row, col)
        oelo, oehi = _edge_tc(gpre, eap, w1c, p['e_W2'], e_b2,
                              p['c_W1'], c_b1, cW2r)
        part = scatter_k(row, oelo, oehi)
        h, x4, v4, pab = _node_tc(part, h, x4, v4, p['v_W1'], v_b1,
                                  vW2r, v_b2, nW1a, nW1b, n_b1,
                                  p['n_W2'], n_b2, W1a, W1b)

    return (x4[:, :3], h, v4[:, :3])
